# baseline (speedup 1.0000x reference)
import functools

import jax
import jax.numpy as jnp
from jax import lax
from jax.experimental import pallas as pl
from jax.experimental.pallas import tpu as pltpu
from jax.experimental.pallas import tpu_sc as plsc

NC = 2
NS = 16
NW = NC * NS
NL = 16

B = 4096
L = 50
D = 64
HIDDEN = 128
OUT = 64
V = 1000000

BPW = B // NW
NBUF = 4
NH = HIDDEN // NL

BLKP = 8192
GRIDP = (V + BLKP - 1) // BLKP
VP = GRIDP * BLKP


def _proj_body(tt_ref, wh_ref, out_ref):
    out_ref[...] = lax.dot_general(
        tt_ref[...], wh_ref[...], (((0,), (0,)), ((), ())),
        preferred_element_type=jnp.float32)


def _tc_project(table_t, wh_scaled):
    return pl.pallas_call(
        _proj_body,
        out_shape=jax.ShapeDtypeStruct((VP, HIDDEN), jnp.float32),
        grid=(GRIDP,),
        in_specs=[
            pl.BlockSpec((D, BLKP), lambda i: (0, i)),
            pl.BlockSpec((D, HIDDEN), lambda i: (0, 0)),
        ],
        out_specs=pl.BlockSpec((BLKP, HIDDEN), lambda i: (i, 0)),
    )(table_t, wh_scaled)


def _sc_gather_pool(x, proj):
    mesh = plsc.VectorSubcoreMesh(core_axis_name="c", subcore_axis_name="s")

    @functools.partial(
        pl.kernel,
        mesh=mesh,
        compiler_params=pltpu.CompilerParams(use_tc_tiling_on_sc=True),
        out_type=jax.ShapeDtypeStruct((B, HIDDEN), jnp.float32),
        scratch_types=(
            [pltpu.VMEM((BPW, L), jnp.int32)]
            + [pltpu.VMEM((L, HIDDEN), jnp.float32)] * NBUF
            + [pltpu.VMEM((BPW, HIDDEN), jnp.float32)]
            + [pltpu.SemaphoreType.DMA] * NBUF
        ),
    )
    def sc_kernel(x_hbm, proj_hbm, out_hbm, idx_v, *rest):
        bufs = rest[:NBUF]
        acc_v = rest[NBUF]
        sems = rest[NBUF + 1:]
        wid = lax.axis_index("s") * NC + lax.axis_index("c")
        base = wid * BPW
        pltpu.sync_copy(x_hbm.at[pl.ds(base, BPW)], idx_v)
        for k in range(NBUF - 1):
            pltpu.async_copy(proj_hbm.at[idx_v.at[k]], bufs[k], sems[k])

        def acc_row(r, buf):
            a = [buf[0, pl.ds(dd * NL, NL)] for dd in range(NH)]
            for j in range(1, L):
                for dd in range(NH):
                    a[dd] = a[dd] + buf[j, pl.ds(dd * NL, NL)]
            for dd in range(NH):
                acc_v[r, pl.ds(dd * NL, NL)] = a[dd]

        def step(r, k):
            pltpu.make_async_copy(proj_hbm.at[idx_v.at[r]], bufs[k],
                                  sems[k]).wait()
            acc_row(r, bufs[k])

        def body(g, carry):
            r0 = NBUF * g
            for k in range(NBUF):
                kn = (k + NBUF - 1) % NBUF
                pltpu.async_copy(proj_hbm.at[idx_v.at[r0 + k + NBUF - 1]],
                                 bufs[kn], sems[kn])
                step(r0 + k, k)
            return carry

        lax.fori_loop(0, BPW // NBUF - 1, body, None)
        r0 = BPW - NBUF
        for k in range(NBUF):
            if r0 + k + NBUF - 1 < BPW:
                kn = (k + NBUF - 1) % NBUF
                pltpu.async_copy(proj_hbm.at[idx_v.at[r0 + k + NBUF - 1]],
                                 bufs[kn], sems[kn])
            step(r0 + k, k)
        pltpu.sync_copy(acc_v, out_hbm.at[pl.ds(base, BPW)])

    return sc_kernel(x, proj)


def _mlp_body(pooled_ref, bh_ref, wo_ref, bo_ref, out_ref):
    h = jnp.tanh(pooled_ref[...] + bh_ref[...])
    out_ref[...] = jnp.tanh(
        jnp.dot(h, wo_ref[...], preferred_element_type=jnp.float32)
        + bo_ref[...]
    )


def _tc_mlp(pooled, bh, wo_t, bo):
    blk = 2048
    return pl.pallas_call(
        _mlp_body,
        out_shape=jax.ShapeDtypeStruct((B, OUT), jnp.float32),
        grid=(B // blk,),
        in_specs=[
            pl.BlockSpec((blk, HIDDEN), lambda i: (i, 0)),
            pl.BlockSpec((1, HIDDEN), lambda i: (0, 0)),
            pl.BlockSpec((HIDDEN, OUT), lambda i: (0, 0)),
            pl.BlockSpec((1, OUT), lambda i: (0, 0)),
        ],
        out_specs=pl.BlockSpec((blk, OUT), lambda i: (i, 0)),
    )(pooled, bh, wo_t, bo)


def kernel(x, table, W_h, b_h, W_o, b_o):
    wh_scaled = W_h.T * (1.0 / L)
    proj = _tc_project(table.T, wh_scaled)
    pooled = _sc_gather_pool(x, proj)
    return _tc_mlp(pooled, b_h[None, :], W_o.T, b_o[None, :])


# speedup vs baseline: 1.2501x; 1.2501x over previous
import functools

import jax
import jax.numpy as jnp
from jax import lax
from jax.experimental import pallas as pl
from jax.experimental.pallas import tpu as pltpu
from jax.experimental.pallas import tpu_sc as plsc

NC = 2
NS = 16
NW = NC * NS
NL = 16

B = 4096
L = 50
D = 64
HIDDEN = 128
OUT = 64
V = 1000000

BPW = B // NW
NBUF = 4
UNROLL = 10
NH = HIDDEN // NL

BLKP = 8192
GRIDP = (V + BLKP - 1) // BLKP
VP = GRIDP * BLKP


def _proj_body(tt_ref, wh_ref, out_ref):
    out_ref[...] = lax.dot_general(
        tt_ref[...], wh_ref[...], (((0,), (0,)), ((), ())),
        preferred_element_type=jnp.float32)


def _tc_project(table_t, wh_scaled):
    return pl.pallas_call(
        _proj_body,
        out_shape=jax.ShapeDtypeStruct((VP, HIDDEN), jnp.float32),
        grid=(GRIDP,),
        in_specs=[
            pl.BlockSpec((D, BLKP), lambda i: (0, i)),
            pl.BlockSpec((D, HIDDEN), lambda i: (0, 0)),
        ],
        out_specs=pl.BlockSpec((BLKP, HIDDEN), lambda i: (i, 0)),
    )(table_t, wh_scaled)


def _sc_gather_pool(x, proj):
    mesh = plsc.VectorSubcoreMesh(core_axis_name="c", subcore_axis_name="s")

    @functools.partial(
        pl.kernel,
        mesh=mesh,
        compiler_params=pltpu.CompilerParams(use_tc_tiling_on_sc=True),
        out_type=jax.ShapeDtypeStruct((B, HIDDEN), jnp.float32),
        scratch_types=(
            [pltpu.VMEM((BPW, L), jnp.int32)]
            + [pltpu.VMEM((L, HIDDEN), jnp.float32)] * NBUF
            + [pltpu.VMEM((BPW, HIDDEN), jnp.float32)]
            + [pltpu.SemaphoreType.DMA] * NBUF
        ),
    )
    def sc_kernel(x_hbm, proj_hbm, out_hbm, idx_v, *rest):
        bufs = rest[:NBUF]
        acc_v = rest[NBUF]
        sems = rest[NBUF + 1:]
        wid = lax.axis_index("s") * NC + lax.axis_index("c")
        base = wid * BPW
        pltpu.sync_copy(x_hbm.at[pl.ds(base, BPW)], idx_v)
        for k in range(NBUF - 1):
            pltpu.async_copy(proj_hbm.at[idx_v.at[k]], bufs[k], sems[k])

        def acc_row(r, buf):
            def jbody(jb, accs):
                a = list(accs)
                for u in range(UNROLL):
                    for dd in range(NH):
                        a[dd] = a[dd] + buf[jb * UNROLL + u,
                                            pl.ds(dd * NL, NL)]
                return tuple(a)

            init = tuple(jnp.zeros((NL,), jnp.float32) for _ in range(NH))
            a = lax.fori_loop(0, L // UNROLL, jbody, init)
            for dd in range(NH):
                acc_v[r, pl.ds(dd * NL, NL)] = a[dd]

        def step(r, k):
            pltpu.make_async_copy(proj_hbm.at[idx_v.at[r]], bufs[k],
                                  sems[k]).wait()
            acc_row(r, bufs[k])

        def body(g, carry):
            r0 = NBUF * g
            for k in range(NBUF):
                kn = (k + NBUF - 1) % NBUF
                pltpu.async_copy(proj_hbm.at[idx_v.at[r0 + k + NBUF - 1]],
                                 bufs[kn], sems[kn])
                step(r0 + k, k)
            return carry

        lax.fori_loop(0, BPW // NBUF - 1, body, None)
        r0 = BPW - NBUF
        for k in range(NBUF):
            if r0 + k + NBUF - 1 < BPW:
                kn = (k + NBUF - 1) % NBUF
                pltpu.async_copy(proj_hbm.at[idx_v.at[r0 + k + NBUF - 1]],
                                 bufs[kn], sems[kn])
            step(r0 + k, k)
        pltpu.sync_copy(acc_v, out_hbm.at[pl.ds(base, BPW)])

    return sc_kernel(x, proj)


def _mlp_body(pooled_ref, bh_ref, wo_ref, bo_ref, out_ref):
    h = jnp.tanh(pooled_ref[...] + bh_ref[...])
    out_ref[...] = jnp.tanh(
        jnp.dot(h, wo_ref[...], preferred_element_type=jnp.float32)
        + bo_ref[...]
    )


def _tc_mlp(pooled, bh, wo_t, bo):
    blk = 2048
    return pl.pallas_call(
        _mlp_body,
        out_shape=jax.ShapeDtypeStruct((B, OUT), jnp.float32),
        grid=(B // blk,),
        in_specs=[
            pl.BlockSpec((blk, HIDDEN), lambda i: (i, 0)),
            pl.BlockSpec((1, HIDDEN), lambda i: (0, 0)),
            pl.BlockSpec((HIDDEN, OUT), lambda i: (0, 0)),
            pl.BlockSpec((1, OUT), lambda i: (0, 0)),
        ],
        out_specs=pl.BlockSpec((blk, OUT), lambda i: (i, 0)),
    )(pooled, bh, wo_t, bo)


def kernel(x, table, W_h, b_h, W_o, b_o):
    wh_scaled = W_h.T * (1.0 / L)
    proj = _tc_project(table.T, wh_scaled)
    pooled = _sc_gather_pool(x, proj)
    return _tc_mlp(pooled, b_h[None, :], W_o.T, b_o[None, :])


# speedup vs baseline: 1.3433x; 1.0746x over previous
import functools

import jax
import jax.numpy as jnp
from jax import lax
from jax.experimental import pallas as pl
from jax.experimental.pallas import tpu as pltpu
from jax.experimental.pallas import tpu_sc as plsc

NC = 2
NS = 16
NW = NC * NS
NL = 16

B = 4096
L = 50
D = 64
HIDDEN = 128
OUT = 64
V = 1000000

BPW = B // NW
NBUF = 4
UNROLL = 10
NH = HIDDEN // NL

BLKP = 16384
GRIDP = (V + BLKP - 1) // BLKP
VP = GRIDP * BLKP


def _proj_body(tt_ref, wh_ref, out_ref):
    out_ref[...] = lax.dot_general(
        tt_ref[...], wh_ref[...], (((0,), (0,)), ((), ())),
        preferred_element_type=jnp.float32)


def _tc_project(table_t, wh_scaled):
    return pl.pallas_call(
        _proj_body,
        out_shape=jax.ShapeDtypeStruct((VP, HIDDEN), jnp.float32),
        grid=(GRIDP,),
        in_specs=[
            pl.BlockSpec((D, BLKP), lambda i: (0, i)),
            pl.BlockSpec((D, HIDDEN), lambda i: (0, 0)),
        ],
        out_specs=pl.BlockSpec((BLKP, HIDDEN), lambda i: (i, 0)),
    )(table_t, wh_scaled)


def _sc_gather_pool(x, proj):
    mesh = plsc.VectorSubcoreMesh(core_axis_name="c", subcore_axis_name="s")

    @functools.partial(
        pl.kernel,
        mesh=mesh,
        compiler_params=pltpu.CompilerParams(use_tc_tiling_on_sc=True),
        out_type=jax.ShapeDtypeStruct((B, HIDDEN), jnp.float32),
        scratch_types=(
            [pltpu.VMEM((BPW, L), jnp.int32)]
            + [pltpu.VMEM((L, HIDDEN), jnp.float32)] * NBUF
            + [pltpu.VMEM((BPW, HIDDEN), jnp.float32)]
            + [pltpu.SemaphoreType.DMA] * NBUF
        ),
    )
    def sc_kernel(x_hbm, proj_hbm, out_hbm, idx_v, *rest):
        bufs = rest[:NBUF]
        acc_v = rest[NBUF]
        sems = rest[NBUF + 1:]
        wid = lax.axis_index("s") * NC + lax.axis_index("c")
        base = wid * BPW
        pltpu.sync_copy(x_hbm.at[pl.ds(base, BPW)], idx_v)
        for k in range(NBUF - 1):
            pltpu.async_copy(proj_hbm.at[idx_v.at[k]], bufs[k], sems[k])

        def acc_row(r, buf):
            def jbody(jb, accs):
                a = list(accs)
                for u in range(UNROLL):
                    for dd in range(NH):
                        a[dd] = a[dd] + buf[jb * UNROLL + u,
                                            pl.ds(dd * NL, NL)]
                return tuple(a)

            init = tuple(jnp.zeros((NL,), jnp.float32) for _ in range(NH))
            a = lax.fori_loop(0, L // UNROLL, jbody, init)
            for dd in range(NH):
                acc_v[r, pl.ds(dd * NL, NL)] = a[dd]

        def step(r, k):
            pltpu.make_async_copy(proj_hbm.at[idx_v.at[r]], bufs[k],
                                  sems[k]).wait()
            acc_row(r, bufs[k])

        def body(g, carry):
            r0 = NBUF * g
            for k in range(NBUF):
                kn = (k + NBUF - 1) % NBUF
                pltpu.async_copy(proj_hbm.at[idx_v.at[r0 + k + NBUF - 1]],
                                 bufs[kn], sems[kn])
                step(r0 + k, k)
            return carry

        lax.fori_loop(0, BPW // NBUF - 1, body, None)
        r0 = BPW - NBUF
        for k in range(NBUF):
            if r0 + k + NBUF - 1 < BPW:
                kn = (k + NBUF - 1) % NBUF
                pltpu.async_copy(proj_hbm.at[idx_v.at[r0 + k + NBUF - 1]],
                                 bufs[kn], sems[kn])
            step(r0 + k, k)
        pltpu.sync_copy(acc_v, out_hbm.at[pl.ds(base, BPW)])

    return sc_kernel(x, proj)


def _mlp_body(pooled_ref, bh_ref, wo_ref, bo_ref, out_ref):
    h = jnp.tanh(pooled_ref[...] + bh_ref[...])
    out_ref[...] = jnp.tanh(
        jnp.dot(h, wo_ref[...], preferred_element_type=jnp.float32)
        + bo_ref[...]
    )


def _tc_mlp(pooled, bh, wo_t, bo):
    blk = 2048
    return pl.pallas_call(
        _mlp_body,
        out_shape=jax.ShapeDtypeStruct((B, OUT), jnp.float32),
        grid=(B // blk,),
        in_specs=[
            pl.BlockSpec((blk, HIDDEN), lambda i: (i, 0)),
            pl.BlockSpec((1, HIDDEN), lambda i: (0, 0)),
            pl.BlockSpec((HIDDEN, OUT), lambda i: (0, 0)),
            pl.BlockSpec((1, OUT), lambda i: (0, 0)),
        ],
        out_specs=pl.BlockSpec((blk, OUT), lambda i: (i, 0)),
    )(pooled, bh, wo_t, bo)


def kernel(x, table, W_h, b_h, W_o, b_o):
    wh_scaled = W_h.T * (1.0 / L)
    proj = _tc_project(table.T, wh_scaled)
    pooled = _sc_gather_pool(x, proj)
    return _tc_mlp(pooled, b_h[None, :], W_o.T, b_o[None, :])


# speedup vs baseline: 1.3703x; 1.0201x over previous
import functools

import jax
import jax.numpy as jnp
from jax import lax
from jax.experimental import pallas as pl
from jax.experimental.pallas import tpu as pltpu
from jax.experimental.pallas import tpu_sc as plsc

NC = 2
NS = 16
NW = NC * NS
NL = 16

B = 4096
L = 50
D = 64
HIDDEN = 128
OUT = 64
V = 1000000

BPW = B // NW
NBUF = 4
UNROLL = 10
NH = HIDDEN // NL

BLKP = 32768
GRIDP = (V + BLKP - 1) // BLKP
VP = GRIDP * BLKP


def _proj_body(tt_ref, wh_ref, out_ref):
    out_ref[...] = lax.dot_general(
        tt_ref[...], wh_ref[...], (((0,), (0,)), ((), ())),
        preferred_element_type=jnp.float32)


def _tc_project(table_t, wh_scaled):
    return pl.pallas_call(
        _proj_body,
        out_shape=jax.ShapeDtypeStruct((VP, HIDDEN), jnp.float32),
        grid=(GRIDP,),
        in_specs=[
            pl.BlockSpec((D, BLKP), lambda i: (0, i)),
            pl.BlockSpec((D, HIDDEN), lambda i: (0, 0)),
        ],
        out_specs=pl.BlockSpec((BLKP, HIDDEN), lambda i: (i, 0)),
    )(table_t, wh_scaled)


def _sc_gather_pool(x, proj):
    mesh = plsc.VectorSubcoreMesh(core_axis_name="c", subcore_axis_name="s")

    @functools.partial(
        pl.kernel,
        mesh=mesh,
        compiler_params=pltpu.CompilerParams(use_tc_tiling_on_sc=True),
        out_type=jax.ShapeDtypeStruct((B, HIDDEN), jnp.float32),
        scratch_types=(
            [pltpu.VMEM((BPW, L), jnp.int32)]
            + [pltpu.VMEM((L, HIDDEN), jnp.float32)] * NBUF
            + [pltpu.VMEM((BPW, HIDDEN), jnp.float32)]
            + [pltpu.SemaphoreType.DMA] * NBUF
        ),
    )
    def sc_kernel(x_hbm, proj_hbm, out_hbm, idx_v, *rest):
        bufs = rest[:NBUF]
        acc_v = rest[NBUF]
        sems = rest[NBUF + 1:]
        wid = lax.axis_index("s") * NC + lax.axis_index("c")
        base = wid * BPW
        pltpu.sync_copy(x_hbm.at[pl.ds(base, BPW)], idx_v)
        for k in range(NBUF - 1):
            pltpu.async_copy(proj_hbm.at[idx_v.at[k]], bufs[k], sems[k])

        def acc_row(r, buf):
            def jbody(jb, accs):
                a = list(accs)
                for u in range(UNROLL):
                    for dd in range(NH):
                        a[dd] = a[dd] + buf[jb * UNROLL + u,
                                            pl.ds(dd * NL, NL)]
                return tuple(a)

            init = tuple(jnp.zeros((NL,), jnp.float32) for _ in range(NH))
            a = lax.fori_loop(0, L // UNROLL, jbody, init)
            for dd in range(NH):
                acc_v[r, pl.ds(dd * NL, NL)] = a[dd]

        def step(r, k):
            pltpu.make_async_copy(proj_hbm.at[idx_v.at[r]], bufs[k],
                                  sems[k]).wait()
            acc_row(r, bufs[k])

        def body(g, carry):
            r0 = NBUF * g
            for k in range(NBUF):
                kn = (k + NBUF - 1) % NBUF
                pltpu.async_copy(proj_hbm.at[idx_v.at[r0 + k + NBUF - 1]],
                                 bufs[kn], sems[kn])
                step(r0 + k, k)
            return carry

        lax.fori_loop(0, BPW // NBUF - 1, body, None)
        r0 = BPW - NBUF
        for k in range(NBUF):
            if r0 + k + NBUF - 1 < BPW:
                kn = (k + NBUF - 1) % NBUF
                pltpu.async_copy(proj_hbm.at[idx_v.at[r0 + k + NBUF - 1]],
                                 bufs[kn], sems[kn])
            step(r0 + k, k)
        pltpu.sync_copy(acc_v, out_hbm.at[pl.ds(base, BPW)])

    return sc_kernel(x, proj)


def _mlp_body(pooled_ref, bh_ref, wo_ref, bo_ref, out_ref):
    h = jnp.tanh(pooled_ref[...] + bh_ref[...])
    out_ref[...] = jnp.tanh(
        jnp.dot(h, wo_ref[...], preferred_element_type=jnp.float32)
        + bo_ref[...]
    )


def _tc_mlp(pooled, bh, wo_t, bo):
    blk = 2048
    return pl.pallas_call(
        _mlp_body,
        out_shape=jax.ShapeDtypeStruct((B, OUT), jnp.float32),
        grid=(B // blk,),
        in_specs=[
            pl.BlockSpec((blk, HIDDEN), lambda i: (i, 0)),
            pl.BlockSpec((1, HIDDEN), lambda i: (0, 0)),
            pl.BlockSpec((HIDDEN, OUT), lambda i: (0, 0)),
            pl.BlockSpec((1, OUT), lambda i: (0, 0)),
        ],
        out_specs=pl.BlockSpec((blk, OUT), lambda i: (i, 0)),
    )(pooled, bh, wo_t, bo)


def kernel(x, table, W_h, b_h, W_o, b_o):
    wh_scaled = W_h.T * (1.0 / L)
    proj = _tc_project(table.T, wh_scaled)
    pooled = _sc_gather_pool(x, proj)
    return _tc_mlp(pooled, b_h[None, :], W_o.T, b_o[None, :])


# speedup vs baseline: 1.4039x; 1.0245x over previous
import functools

import jax
import jax.numpy as jnp
from jax import lax
from jax.experimental import pallas as pl
from jax.experimental.pallas import tpu as pltpu
from jax.experimental.pallas import tpu_sc as plsc

NC = 2
NS = 16
NW = NC * NS
NL = 16

B = 4096
L = 50
D = 64
HIDDEN = 128
OUT = 64
V = 1000000

BPW = B // NW
NBUF = 8
UNROLL = 10
NH = HIDDEN // NL

BLKP = 32768
GRIDP = (V + BLKP - 1) // BLKP
VP = GRIDP * BLKP


def _proj_body(tt_ref, wh_ref, out_ref):
    out_ref[...] = lax.dot_general(
        tt_ref[...], wh_ref[...], (((0,), (0,)), ((), ())),
        preferred_element_type=jnp.float32)


def _tc_project(table_t, wh_scaled):
    return pl.pallas_call(
        _proj_body,
        out_shape=jax.ShapeDtypeStruct((VP, HIDDEN), jnp.float32),
        grid=(GRIDP,),
        in_specs=[
            pl.BlockSpec((D, BLKP), lambda i: (0, i)),
            pl.BlockSpec((D, HIDDEN), lambda i: (0, 0)),
        ],
        out_specs=pl.BlockSpec((BLKP, HIDDEN), lambda i: (i, 0)),
    )(table_t, wh_scaled)


def _sc_gather_pool(x, proj):
    mesh = plsc.VectorSubcoreMesh(core_axis_name="c", subcore_axis_name="s")

    @functools.partial(
        pl.kernel,
        mesh=mesh,
        compiler_params=pltpu.CompilerParams(use_tc_tiling_on_sc=True),
        out_type=jax.ShapeDtypeStruct((B, HIDDEN), jnp.float32),
        scratch_types=(
            [pltpu.VMEM((BPW, L), jnp.int32)]
            + [pltpu.VMEM((L, HIDDEN), jnp.float32)] * NBUF
            + [pltpu.VMEM((BPW, HIDDEN), jnp.float32)]
            + [pltpu.SemaphoreType.DMA] * NBUF
        ),
    )
    def sc_kernel(x_hbm, proj_hbm, out_hbm, idx_v, *rest):
        bufs = rest[:NBUF]
        acc_v = rest[NBUF]
        sems = rest[NBUF + 1:]
        wid = lax.axis_index("s") * NC + lax.axis_index("c")
        base = wid * BPW
        pltpu.sync_copy(x_hbm.at[pl.ds(base, BPW)], idx_v)
        for k in range(NBUF - 1):
            pltpu.async_copy(proj_hbm.at[idx_v.at[k]], bufs[k], sems[k])

        def acc_row(r, buf):
            def jbody(jb, accs):
                a = list(accs)
                for u in range(UNROLL):
                    for dd in range(NH):
                        a[dd] = a[dd] + buf[jb * UNROLL + u,
                                            pl.ds(dd * NL, NL)]
                return tuple(a)

            init = tuple(jnp.zeros((NL,), jnp.float32) for _ in range(NH))
            a = lax.fori_loop(0, L // UNROLL, jbody, init)
            for dd in range(NH):
                acc_v[r, pl.ds(dd * NL, NL)] = a[dd]

        def step(r, k):
            pltpu.make_async_copy(proj_hbm.at[idx_v.at[r]], bufs[k],
                                  sems[k]).wait()
            acc_row(r, bufs[k])

        def body(g, carry):
            r0 = NBUF * g
            for k in range(NBUF):
                kn = (k + NBUF - 1) % NBUF
                pltpu.async_copy(proj_hbm.at[idx_v.at[r0 + k + NBUF - 1]],
                                 bufs[kn], sems[kn])
                step(r0 + k, k)
            return carry

        lax.fori_loop(0, BPW // NBUF - 1, body, None)
        r0 = BPW - NBUF
        for k in range(NBUF):
            if r0 + k + NBUF - 1 < BPW:
                kn = (k + NBUF - 1) % NBUF
                pltpu.async_copy(proj_hbm.at[idx_v.at[r0 + k + NBUF - 1]],
                                 bufs[kn], sems[kn])
            step(r0 + k, k)
        pltpu.sync_copy(acc_v, out_hbm.at[pl.ds(base, BPW)])

    return sc_kernel(x, proj)


def _mlp_body(pooled_ref, bh_ref, wo_ref, bo_ref, out_ref):
    h = jnp.tanh(pooled_ref[...] + bh_ref[...])
    out_ref[...] = jnp.tanh(
        jnp.dot(h, wo_ref[...], preferred_element_type=jnp.float32)
        + bo_ref[...]
    )


def _tc_mlp(pooled, bh, wo_t, bo):
    blk = 2048
    return pl.pallas_call(
        _mlp_body,
        out_shape=jax.ShapeDtypeStruct((B, OUT), jnp.float32),
        grid=(B // blk,),
        in_specs=[
            pl.BlockSpec((blk, HIDDEN), lambda i: (i, 0)),
            pl.BlockSpec((1, HIDDEN), lambda i: (0, 0)),
            pl.BlockSpec((HIDDEN, OUT), lambda i: (0, 0)),
            pl.BlockSpec((1, OUT), lambda i: (0, 0)),
        ],
        out_specs=pl.BlockSpec((blk, OUT), lambda i: (i, 0)),
    )(pooled, bh, wo_t, bo)


def kernel(x, table, W_h, b_h, W_o, b_o):
    wh_scaled = W_h.T * (1.0 / L)
    proj = _tc_project(table.T, wh_scaled)
    pooled = _sc_gather_pool(x, proj)
    return _tc_mlp(pooled, b_h[None, :], W_o.T, b_o[None, :])
